# Initial kernel scaffold; baseline (speedup 1.0000x reference)
import jax
import jax.numpy as jnp
from jax.experimental import pallas as pl


def kernel(atom_embed, node_embed, atom_to_res_idx, atom_mask, W):
    raise NotImplementedError("write your pallas kernel here")


# speedup vs baseline: 5.2997x; 5.2997x over previous
import jax
import jax.numpy as jnp
from jax import lax
from jax.experimental import pallas as pl
from jax.experimental.pallas import tpu as pltpu

_B, _A, _R = 4, 16384, 1024
_C_ATOM, _C_S = 128, 384
_TA = 2048
_AB = _A // _TA


def _body(idx_ref, mask_ref, x_ref, w_ref, node_ref, out_ref,
          acc_ref, cnt_ref, den_ref):
    a = pl.program_id(1)

    x = x_ref[0]
    w = w_ref[...]
    vals = lax.dot_general(x, w, (((1,), (1,)), ((), ())),
                           preferred_element_type=jnp.float32)
    vals = jnp.maximum(vals, 0.0)

    idx_row = idx_ref[0]
    mask_row = mask_ref[0]
    rows = lax.broadcasted_iota(jnp.int32, (_R, _TA), 0)
    onehot = (rows == idx_row).astype(jnp.float32)
    onehotm = onehot * mask_row

    cnt_blk = jnp.sum(onehot, axis=1, keepdims=True)
    den_blk = jnp.sum(onehotm, axis=1, keepdims=True)

    sum_blk = lax.dot_general(
        onehotm.astype(jnp.bfloat16), vals.astype(jnp.bfloat16),
        (((1,), (0,)), ((), ())),
        preferred_element_type=jnp.float32)

    @pl.when(a == 0)
    def _init():
        acc_ref[...] = sum_blk
        cnt_ref[...] = cnt_blk
        den_ref[...] = den_blk

    @pl.when(a > 0)
    def _accumulate():
        acc_ref[...] += sum_blk
        cnt_ref[...] += cnt_blk
        den_ref[...] += den_blk

    @pl.when(a == _AB - 1)
    def _finish():
        c = cnt_ref[...]
        d = den_ref[...]
        out_ref[0] = acc_ref[...] / ((c + 1.0) * d) + node_ref[0]


def kernel(atom_embed, node_embed, atom_to_res_idx, atom_mask, W):
    idx = atom_to_res_idx.astype(jnp.int32).reshape(_B * _AB, 1, _TA)
    mask = atom_mask.reshape(_B * _AB, 1, _TA)
    return pl.pallas_call(
        _body,
        grid=(_B, _AB),
        in_specs=[
            pl.BlockSpec((1, 1, _TA), lambda b, a: (b * _AB + a, 0, 0)),
            pl.BlockSpec((1, 1, _TA), lambda b, a: (b * _AB + a, 0, 0)),
            pl.BlockSpec((1, _TA, _C_ATOM), lambda b, a: (b, a, 0)),
            pl.BlockSpec((_C_S, _C_ATOM), lambda b, a: (0, 0)),
            pl.BlockSpec((1, _R, _C_S), lambda b, a: (b, 0, 0)),
        ],
        out_specs=pl.BlockSpec((1, _R, _C_S), lambda b, a: (b, 0, 0)),
        out_shape=jax.ShapeDtypeStruct((_B, _R, _C_S), jnp.float32),
        scratch_shapes=[
            pltpu.VMEM((_R, _C_S), jnp.float32),
            pltpu.VMEM((_R, 1), jnp.float32),
            pltpu.VMEM((_R, 1), jnp.float32),
        ],
        compiler_params=pltpu.CompilerParams(
            dimension_semantics=("parallel", "arbitrary")),
    )(idx, mask, atom_embed, W, node_embed)


# speedup vs baseline: 5.3012x; 1.0003x over previous
import jax
import jax.numpy as jnp
from jax import lax
from jax.experimental import pallas as pl
from jax.experimental.pallas import tpu as pltpu

_B, _A, _R = 4, 16384, 1024
_C_ATOM, _C_S = 128, 384
_TA = 2048
_AB = _A // _TA


def _body(idx_ref, mask_ref, x_ref, w_ref, node_ref, out_ref,
          acc_ref, cnt_ref, den_ref):
    a = pl.program_id(1)

    x = x_ref[0].astype(jnp.bfloat16)
    w = w_ref[...].astype(jnp.bfloat16)
    vals = lax.dot_general(x, w, (((1,), (1,)), ((), ())),
                           preferred_element_type=jnp.float32)
    vals = jnp.maximum(vals, 0.0)

    idx_row = idx_ref[0]
    mask_row = mask_ref[0]
    rows = lax.broadcasted_iota(jnp.int32, (_R, _TA), 0)
    onehot = (rows == idx_row).astype(jnp.float32)
    onehotm = onehot * mask_row

    cnt_blk = jnp.sum(onehot, axis=1, keepdims=True)
    den_blk = jnp.sum(onehotm, axis=1, keepdims=True)

    sum_blk = lax.dot_general(
        onehotm.astype(jnp.bfloat16), vals.astype(jnp.bfloat16),
        (((1,), (0,)), ((), ())),
        preferred_element_type=jnp.float32)

    @pl.when(a == 0)
    def _init():
        acc_ref[...] = sum_blk
        cnt_ref[...] = cnt_blk
        den_ref[...] = den_blk

    @pl.when(a > 0)
    def _accumulate():
        acc_ref[...] += sum_blk
        cnt_ref[...] += cnt_blk
        den_ref[...] += den_blk

    @pl.when(a == _AB - 1)
    def _finish():
        c = cnt_ref[...]
        d = den_ref[...]
        out_ref[0] = acc_ref[...] / ((c + 1.0) * d) + node_ref[0]


def kernel(atom_embed, node_embed, atom_to_res_idx, atom_mask, W):
    idx = atom_to_res_idx.astype(jnp.int32).reshape(_B * _AB, 1, _TA)
    mask = atom_mask.reshape(_B * _AB, 1, _TA)
    return pl.pallas_call(
        _body,
        grid=(_B, _AB),
        in_specs=[
            pl.BlockSpec((1, 1, _TA), lambda b, a: (b * _AB + a, 0, 0)),
            pl.BlockSpec((1, 1, _TA), lambda b, a: (b * _AB + a, 0, 0)),
            pl.BlockSpec((1, _TA, _C_ATOM), lambda b, a: (b, a, 0)),
            pl.BlockSpec((_C_S, _C_ATOM), lambda b, a: (0, 0)),
            pl.BlockSpec((1, _R, _C_S), lambda b, a: (b, 0, 0)),
        ],
        out_specs=pl.BlockSpec((1, _R, _C_S), lambda b, a: (b, 0, 0)),
        out_shape=jax.ShapeDtypeStruct((_B, _R, _C_S), jnp.float32),
        scratch_shapes=[
            pltpu.VMEM((_R, _C_S), jnp.float32),
            pltpu.VMEM((_R, 1), jnp.float32),
            pltpu.VMEM((_R, 1), jnp.float32),
        ],
        compiler_params=pltpu.CompilerParams(
            dimension_semantics=("parallel", "arbitrary")),
    )(idx, mask, atom_embed, W, node_embed)
